# baseline (speedup 1.0000x reference)
import functools

import jax
import jax.numpy as jnp
from jax import lax
from jax.experimental import pallas as pl
from jax.experimental.pallas import tpu as pltpu
from jax.experimental.pallas import tpu_sc as plsc

_N = 100000
_H = 128
_B = 64
_M = 3
_NW = 32
_CHUNK = _N // _NW
_TN = 240
_NT = 13
_TAIL = _CHUNK - _NT * _TN
_GPAD = 3200
_AW = 144


def _sc_body(nodes_hbm, gid_hbm, q_hbm, out_hbm, tile_v, tail_v, gid_v, q_v, acc_v):
    wid = lax.axis_index("s") * 2 + lax.axis_index("c")

    def zero_g(g, carry):
        for k in range(_AW // 16):
            acc_v[g, pl.ds(16 * k, 16)] = jnp.zeros((16,), jnp.float32)
        return carry

    lax.fori_loop(0, _B, zero_g, 0)

    pltpu.sync_copy(q_hbm, q_v)
    pltpu.sync_copy(gid_hbm.at[wid], gid_v)

    lanes = lax.iota(jnp.int32, 16)

    def lane_sum(s):
        for sh in (1, 2, 4, 8):
            s = s + s.at[lax.rem(lanes + sh, 16)].get(mode="promise_in_bounds")
        return s

    def process_row(buf, local_r, g):
        s = jnp.zeros((16,), jnp.float32)
        rowvecs = []
        for k in range(8):
            rv = buf[local_r, pl.ds(16 * k, 16)]
            qv = q_v[g, pl.ds(16 * k, 16)]
            s = s + rv * qv
            rowvecs.append(rv)
        p = jnp.exp(lane_sum(s))
        for k in range(8):
            plsc.addupdate(acc_v.at[g, pl.ds(16 * k, 16)], rowvecs[k] * p)
        plsc.addupdate(acc_v.at[g, pl.ds(128, 16)], p)

    def group_body(buf, local0, gv):
        g0 = gv[0]

        def fast():
            qv = [q_v[g0, pl.ds(16 * k, 16)] for k in range(8)]
            racc = [jnp.zeros((16,), jnp.float32) for _ in range(8)]
            pacc = jnp.zeros((16,), jnp.float32)
            for r in range(16):
                row = [buf[local0 + r, pl.ds(16 * k, 16)] for k in range(8)]
                s = row[0] * qv[0]
                for k in range(1, 8):
                    s = s + row[k] * qv[k]
                p = jnp.exp(lane_sum(s))
                pacc = pacc + p
                for k in range(8):
                    racc[k] = racc[k] + row[k] * p
            for k in range(8):
                plsc.addupdate(acc_v.at[g0, pl.ds(16 * k, 16)], racc[k])
            plsc.addupdate(acc_v.at[g0, pl.ds(128, 16)], pacc)

        def slow():
            for r in range(16):
                process_row(buf, local0 + r, gv[r])

        lax.cond(g0 == gv[15], fast, slow)

    def tile_body(t, carry):
        pltpu.sync_copy(nodes_hbm.at[wid, pl.ds(t * _TN, _TN)], tile_v)

        def grp_body(grp, c2):
            gv = gid_v[pl.ds(t * _TN + grp * 16, 16)]
            group_body(tile_v, grp * 16, gv)
            return c2

        lax.fori_loop(0, _TN // 16, grp_body, 0)
        return carry

    lax.fori_loop(0, _NT, tile_body, 0)

    pltpu.sync_copy(nodes_hbm.at[wid, pl.ds(_NT * _TN, _TAIL)],
                    tail_v.at[pl.ds(0, _TAIL)])

    def tail_body(r, c2):
        g = gid_v[pl.ds(_NT * _TN + r, 16)][0]
        process_row(tail_v, r, g)
        return c2

    lax.fori_loop(0, _TAIL, tail_body, 0)

    pltpu.sync_copy(acc_v, out_hbm.at[wid])


_sc_pass = functools.partial(
    pl.kernel,
    out_type=jax.ShapeDtypeStruct((_NW, _B, _AW), jnp.float32),
    mesh=plsc.VectorSubcoreMesh(core_axis_name="c", subcore_axis_name="s"),
    scratch_types=[
        pltpu.VMEM((_TN, _H), jnp.float32),
        pltpu.VMEM((8, _H), jnp.float32),
        pltpu.VMEM((_GPAD,), jnp.int32),
        pltpu.VMEM((_B, _H), jnp.float32),
        pltpu.VMEM((_B, _AW), jnp.float32),
    ],
)(_sc_body)


def _combine(acc_ref):
    tot = acc_ref[0]
    for w in range(1, _NW):
        tot = tot + acc_ref[w]
    racc = tot[:, :_H]
    den = tot[:, _H:_H + 1]
    den_safe = jnp.where(den > 0.0, den, 1.0)
    return racc / den_safe


def _lstm(x, h, c, Wih, Whh, b):
    g = (lax.dot_general(x, Wih, (((1,), (1,)), ((), ())),
                         preferred_element_type=jnp.float32)
         + lax.dot_general(h, Whh, (((1,), (1,)), ((), ())),
                           preferred_element_type=jnp.float32)
         + b)
    i = jax.nn.sigmoid(g[:, 0 * _H:1 * _H])
    f = jax.nn.sigmoid(g[:, 1 * _H:2 * _H])
    gg = jnp.tanh(g[:, 2 * _H:3 * _H])
    o = jax.nn.sigmoid(g[:, 3 * _H:4 * _H])
    c2 = f * c + i * gg
    h2 = o * jnp.tanh(c2)
    return h2, c2


def _step_body(acc_ref, q_ref, h0_ref, c0_ref, h1_ref, c1_ref,
               Wih0_ref, Whh0_ref, b0_ref, Wih1_ref, Whh1_ref, b1_ref,
               qn_ref, h0n_ref, c0n_ref, h1n_ref, c1n_ref):
    r = _combine(acc_ref)
    q_star = jnp.concatenate([q_ref[...], r], axis=1)
    h0n, c0n = _lstm(q_star, h0_ref[...], c0_ref[...],
                     Wih0_ref[...], Whh0_ref[...], b0_ref[...])
    h1n, c1n = _lstm(h0n, h1_ref[...], c1_ref[...],
                     Wih1_ref[...], Whh1_ref[...], b1_ref[...])
    qn_ref[...] = h1n
    h0n_ref[...] = h0n
    c0n_ref[...] = c0n
    h1n_ref[...] = h1n
    c1n_ref[...] = c1n


def _final_body(acc_ref, q_ref, out_ref):
    r = _combine(acc_ref)
    out_ref[...] = jnp.concatenate([q_ref[...], r], axis=1)


_step_call = pl.pallas_call(
    _step_body,
    out_shape=[jax.ShapeDtypeStruct((_B, _H), jnp.float32)] * 5,
)

_final_call = pl.pallas_call(
    _final_body,
    out_shape=jax.ShapeDtypeStruct((_B, 2 * _H), jnp.float32),
)


@jax.jit
def kernel(nodes, graph_id, Wih0, Whh0, bih0, bhh0, Wih1, Whh1, bih1, bhh1):
    nodesR = nodes.reshape(_NW, _CHUNK, _H)
    gidR = jnp.pad(graph_id.reshape(_NW, _CHUNK), ((0, 0), (0, _GPAD - _CHUNK)))
    b0 = (bih0 + bhh0).reshape(1, 4 * _H)
    b1 = (bih1 + bhh1).reshape(1, 4 * _H)
    q = jnp.zeros((_B, _H), jnp.float32)
    h0 = jnp.zeros((_B, _H), jnp.float32)
    c0 = jnp.zeros((_B, _H), jnp.float32)
    h1 = jnp.zeros((_B, _H), jnp.float32)
    c1 = jnp.zeros((_B, _H), jnp.float32)
    out = None
    for i in range(_M):
        acc = _sc_pass(nodesR, gidR, q)
        if i < _M - 1:
            q, h0, c0, h1, c1 = _step_call(acc, q, h0, c0, h1, c1,
                                           Wih0, Whh0, b0, Wih1, Whh1, b1)
        else:
            out = _final_call(acc, q)
    return out


# speedup vs baseline: 2.7114x; 2.7114x over previous
import functools

import jax
import jax.numpy as jnp
from jax.experimental import pallas as pl
from jax.experimental.pallas import tpu as pltpu

_N = 100000
_H = 128
_B = 64
_M = 3
_T = 20000
_NB = _N // _T

_NEG_INF = float("-inf")


def _lstm(x, h, c, Wih, Whh, b):
    g = (jax.lax.dot_general(x, Wih, (((1,), (1,)), ((), ())),
                             preferred_element_type=jnp.float32)
         + jax.lax.dot_general(h, Whh, (((1,), (1,)), ((), ())),
                               preferred_element_type=jnp.float32)
         + b)
    i = jax.nn.sigmoid(g[:, 0 * _H:1 * _H])
    f = jax.nn.sigmoid(g[:, 1 * _H:2 * _H])
    gg = jnp.tanh(g[:, 2 * _H:3 * _H])
    o = jax.nn.sigmoid(g[:, 3 * _H:4 * _H])
    c2 = f * c + i * gg
    h2 = o * jnp.tanh(c2)
    return h2, c2


def _body(nodes_ref, gid_ref, Wih0_ref, Whh0_ref, b0_ref, Wih1_ref,
          Whh1_ref, b1_ref, out_ref,
          den_ref, raccT_ref, q_ref, h0_ref, c0_ref, h1_ref, c1_ref):
    t = pl.program_id(0)

    @pl.when(t == 0)
    def _init():
        den_ref[...] = jnp.zeros((1, _B), jnp.float32)
        raccT_ref[...] = jnp.zeros((_H, _B), jnp.float32)
        q_ref[...] = jnp.zeros((_B, _H), jnp.float32)
        h0_ref[...] = jnp.zeros((_B, _H), jnp.float32)
        c0_ref[...] = jnp.zeros((_B, _H), jnp.float32)
        h1_ref[...] = jnp.zeros((_B, _H), jnp.float32)
        c1_ref[...] = jnp.zeros((_B, _H), jnp.float32)

    @pl.when(jnp.logical_and(t > 0, t % _NB == 0))
    def _finalize():
        den = den_ref[...]
        den_safe = jnp.where(den > 0.0, den, 1.0)
        rT = raccT_ref[...] / den_safe
        rowi = jax.lax.broadcasted_iota(jnp.int32, (_B, _B), 0)
        coli = jax.lax.broadcasted_iota(jnp.int32, (_B, _B), 1)
        eye = (rowi == coli).astype(jnp.float32)
        r = jax.lax.dot_general(eye, rT, (((1,), (1,)), ((), ())),
                                preferred_element_type=jnp.float32)
        q_star = jnp.concatenate([q_ref[...], r], axis=1)

        @pl.when(t == _M * _NB)
        def _emit():
            out_ref[...] = q_star

        @pl.when(t < _M * _NB)
        def _step_lstm():
            h0n, c0n = _lstm(q_star, h0_ref[...], c0_ref[...],
                             Wih0_ref[...], Whh0_ref[...], b0_ref[...])
            h1n, c1n = _lstm(h0n, h1_ref[...], c1_ref[...],
                             Wih1_ref[...], Whh1_ref[...], b1_ref[...])
            h0_ref[...] = h0n
            c0_ref[...] = c0n
            h1_ref[...] = h1n
            c1_ref[...] = c1n
            q_ref[...] = h1n
            den_ref[...] = jnp.zeros((1, _B), jnp.float32)
            raccT_ref[...] = jnp.zeros((_H, _B), jnp.float32)

    @pl.when(t < _M * _NB)
    def _accumulate():
        blk = nodes_ref[...]
        gid = gid_ref[0, 0, :]
        seg = jax.lax.broadcasted_iota(jnp.int32, (_T, _B), 1)
        mask = gid[:, None] == seg
        e = jax.lax.dot_general(blk, q_ref[...].astype(jnp.bfloat16),
                                (((1,), (1,)), ((), ())),
                                preferred_element_type=jnp.float32)
        p = jnp.where(mask, jnp.exp(e), 0.0)
        den_ref[...] = den_ref[...] + jnp.sum(p, axis=0, keepdims=True)
        raccT_ref[...] = (raccT_ref[...]
                          + jax.lax.dot_general(
                              blk, p.astype(jnp.bfloat16), (((0,), (0,)), ((), ())),
                              preferred_element_type=jnp.float32))


@jax.jit
def kernel(nodes, graph_id, Wih0, Whh0, bih0, bhh0, Wih1, Whh1, bih1, bhh1):
    nodes = nodes.astype(jnp.bfloat16)
    gid3 = graph_id.reshape(_NB, 1, _T)
    b0 = (bih0 + bhh0).reshape(1, 4 * _H)
    b1 = (bih1 + bhh1).reshape(1, 4 * _H)
    grid = (_M * _NB + 1,)
    res = pl.pallas_call(
        _body,
        grid=grid,
        in_specs=[
            pl.BlockSpec((_T, _H), lambda t: (t % _NB, 0)),
            pl.BlockSpec((1, 1, _T), lambda t: (t % _NB, 0, 0)),
            pl.BlockSpec((4 * _H, 2 * _H), lambda t: (0, 0)),
            pl.BlockSpec((4 * _H, _H), lambda t: (0, 0)),
            pl.BlockSpec((1, 4 * _H), lambda t: (0, 0)),
            pl.BlockSpec((4 * _H, _H), lambda t: (0, 0)),
            pl.BlockSpec((4 * _H, _H), lambda t: (0, 0)),
            pl.BlockSpec((1, 4 * _H), lambda t: (0, 0)),
        ],
        out_specs=pl.BlockSpec((_B, 2 * _H), lambda t: (0, 0)),
        out_shape=jax.ShapeDtypeStruct((_B, 2 * _H), jnp.float32),
        scratch_shapes=[
            pltpu.VMEM((1, _B), jnp.float32),
            pltpu.VMEM((_H, _B), jnp.float32),
            pltpu.VMEM((_B, _H), jnp.float32),
            pltpu.VMEM((_B, _H), jnp.float32),
            pltpu.VMEM((_B, _H), jnp.float32),
            pltpu.VMEM((_B, _H), jnp.float32),
            pltpu.VMEM((_B, _H), jnp.float32),
        ],
    )(nodes, gid3, Wih0, Whh0, b0, Wih1, Whh1, b1)
    return res
